# Initial kernel scaffold; baseline (speedup 1.0000x reference)
import functools

import numpy as np
import jax
import jax.numpy as jnp
from jax.experimental import pallas as pl
from jax.experimental.pallas import tpu as pltpu

_N = 10000
_E = 320000
_K = 32
_IN = 128
_HID = 256
_TCH = 128
_OUT = 256
_NP = 10240
_B = 64


def _layer_norm(x, g, b):
    m = jnp.mean(x, axis=-1, keepdims=True)
    v = jnp.mean((x - m) ** 2, axis=-1, keepdims=True)
    return (x - m) * jax.lax.rsqrt(v + 1e-5) * g + b


def _gelu(x):
    return x * 0.5 * (1.0 + jax.lax.erf(x * np.float32(0.7071067811865476)))


def _dot(a, b):
    return jnp.dot(a, b, preferred_element_type=jnp.float32)


def _mixer_body(dt_ref, msk_ref, attr_ref, tew_ref, thwt_ref, thwa_ref,
                thb_ref, tng_ref, tnb_ref, tl1t_ref, tl1b_ref, tl2t_ref,
                tl2b_ref, cng_ref, cnb_ref, cl1_ref, cl1b_ref, cl2_ref,
                cl2b_ref, hng_ref, hnb_ref, hlw_ref, hlb_ref, out_ref):
    r = _K * _B
    dt3 = dt_ref[...][:, :, None]
    msk3 = msk_ref[...][:, :, None]
    tew = tew_ref[...].reshape(1, 1, _TCH)
    te2 = jnp.cos(dt3 * tew).reshape(r, _TCH)
    attr2 = attr_ref[...].reshape(r, _IN)
    mskb = jnp.broadcast_to(msk3, (_K, _B, _HID)).reshape(r, _HID)

    h = _dot(te2, thwt_ref[...]) + _dot(attr2, thwa_ref[...]) + thb_ref[...]
    x = h * mskb

    ln1 = _layer_norm(x, tng_ref[...], tnb_ref[...])
    y = ln1.reshape(_K, _B * _HID)
    tmid = _gelu(_dot(tl1t_ref[...], y) + tl1b_ref[...])
    tout = _dot(tl2t_ref[...], tmid) + tl2b_ref[...]
    h_token = tout.reshape(r, _HID) + x

    ln2 = _layer_norm(h_token, cng_ref[...], cnb_ref[...])
    u = _gelu(_dot(ln2, cl1_ref[...]) + cl1b_ref[...])
    v = _dot(u, cl2_ref[...]) + cl2b_ref[...]
    h_chan = v + h_token

    ln3 = _layer_norm(h_chan, hng_ref[...], hnb_ref[...])
    acc = ln3[0:_B, :]
    for k in range(1, _K):
        acc = acc + ln3[k * _B:(k + 1) * _B, :]
    mean = acc * np.float32(1.0 / _K)
    out_ref[...] = _dot(mean, hlw_ref[...]) + hlb_ref[...]


def _run_mixer(dt_t, msk_t, attr3, te_w, th_W, th_b, tn_g, tn_b, tl1_W,
               tl1_b, tl2_W, tl2_b, cn_g, cn_b, cl1_W, cl1_b, cl2_W, cl2_b,
               hn_g, hn_b, hl_W, hl_b):
    grid = (_NP // _B,)
    full = lambda shape: pl.BlockSpec(shape, lambda i: (0,) * len(shape))
    in_specs = [
        pl.BlockSpec((_K, _B), lambda i: (0, i)),
        pl.BlockSpec((_K, _B), lambda i: (0, i)),
        pl.BlockSpec((_K, _B, _IN), lambda i: (0, i, 0)),
        full((1, _TCH)),
        full((_TCH, _HID)),
        full((_IN, _HID)),
        full((1, _HID)),
        full((1, _HID)), full((1, _HID)),
        full((_K // 2, _K)), full((_K // 2, 1)),
        full((_K, _K // 2)), full((_K, 1)),
        full((1, _HID)), full((1, _HID)),
        full((_HID, 4 * _HID)), full((1, 4 * _HID)),
        full((4 * _HID, _HID)), full((1, _HID)),
        full((1, _HID)), full((1, _HID)),
        full((_HID, _OUT)), full((1, _OUT)),
    ]
    out = pl.pallas_call(
        _mixer_body,
        grid=grid,
        in_specs=in_specs,
        out_specs=pl.BlockSpec((_B, _OUT), lambda i: (i, 0)),
        out_shape=jax.ShapeDtypeStruct((_NP, _OUT), jnp.float32),
    )(dt_t, msk_t, attr3, te_w.reshape(1, _TCH),
      th_W[:_TCH], th_W[_TCH:], th_b.reshape(1, _HID),
      tn_g.reshape(1, _HID), tn_b.reshape(1, _HID),
      tl1_W.T, tl1_b.reshape(_K // 2, 1),
      tl2_W.T, tl2_b.reshape(_K, 1),
      cn_g.reshape(1, _HID), cn_b.reshape(1, _HID),
      cl1_W, cl1_b.reshape(1, 4 * _HID),
      cl2_W, cl2_b.reshape(1, _HID),
      hn_g.reshape(1, _HID), hn_b.reshape(1, _HID),
      hl_W, hl_b.reshape(1, _OUT))
    return out


def kernel(edge_index, edge_attr, edge_time, seed_time, th_W, th_b, tn_g,
           tn_b, tl1_W, tl1_b, tl2_W, tl2_b, cn_g, cn_b, cl1_W, cl1_b,
           cl2_W, cl2_b, hn_g, hn_b, hl_W, hl_b):
    col = edge_index[1]
    t = edge_time
    mask = t <= seed_time[col]
    col_eff = jnp.where(mask, col, _N).astype(jnp.int32)
    perm = jnp.lexsort([-t, col_eff]).astype(jnp.int32)
    cs = col_eff[perm]
    mask_s = mask[perm]
    col_s = col[perm]
    dt = (seed_time[col_s] - t[perm]).astype(jnp.float32)
    rank = (jnp.arange(_E, dtype=jnp.int32)
            - jnp.searchsorted(cs, cs, side="left").astype(jnp.int32))
    valid = mask_s & (rank < _K)
    slot = jnp.where(valid, rank * _NP + cs, _NP * _K)

    eid = jnp.zeros((_NP * _K + 1,), jnp.int32).at[slot].set(
        perm, mode="drop")[:_NP * _K]
    dtd = jnp.zeros((_NP * _K + 1,), jnp.float32).at[slot].set(
        dt, mode="drop")[:_NP * _K]
    mskd = jnp.zeros((_NP * _K + 1,), jnp.float32).at[slot].set(
        1.0, mode="drop")[:_NP * _K]
    attr_t = edge_attr[eid]

    te_w = (1.0 / 10.0 ** jnp.linspace(
        0.0, float(np.sqrt(_TCH)), _TCH)).astype(jnp.float32)

    out = _run_mixer(dtd.reshape(_K, _NP), mskd.reshape(_K, _NP),
                     attr_t.reshape(_K, _NP, _IN), te_w, th_W, th_b,
                     tn_g, tn_b, tl1_W, tl1_b, tl2_W, tl2_b, cn_g, cn_b,
                     cl1_W, cl1_b, cl2_W, cl2_b, hn_g, hn_b, hl_W, hl_b)
    return out[:_N]


# speedup vs baseline: 1.1294x; 1.1294x over previous
import functools

import numpy as np
import jax
import jax.numpy as jnp
from jax.experimental import pallas as pl
from jax.experimental.pallas import tpu as pltpu

_N = 10000
_E = 320000
_K = 32
_IN = 128
_HID = 256
_TCH = 128
_OUT = 256
_NP = 10240
_B = 128


def _layer_norm(x, g, b):
    m = jnp.mean(x, axis=-1, keepdims=True)
    v = jnp.mean((x - m) ** 2, axis=-1, keepdims=True)
    return (x - m) * jax.lax.rsqrt(v + 1e-5) * g + b


def _gelu(x):
    return x * 0.5 * (1.0 + jax.lax.erf(x * np.float32(0.7071067811865476)))


def _dot(a, b):
    return jnp.dot(a, b, preferred_element_type=jnp.float32)


def _mixer_body(dt_ref, msk_ref, attr_ref, tew_ref, thwt_ref, thwa_ref,
                thb_ref, tng_ref, tnb_ref, tl1t_ref, tl1b_ref, tl2t_ref,
                tl2b_ref, cng_ref, cnb_ref, cl1_ref, cl1b_ref, cl2_ref,
                cl2b_ref, hng_ref, hnb_ref, hlw_ref, hlb_ref, out_ref):
    r = _K * _B
    dt3 = dt_ref[...][:, :, None]
    msk3 = msk_ref[...][:, :, None]
    tew = tew_ref[...].reshape(1, 1, _TCH)
    te2 = jnp.cos(dt3 * tew).reshape(r, _TCH)
    attr2 = attr_ref[...].reshape(r, _IN)
    mskb = jnp.broadcast_to(msk3, (_K, _B, _HID)).reshape(r, _HID)

    h = _dot(te2, thwt_ref[...]) + _dot(attr2, thwa_ref[...]) + thb_ref[...]
    x = h * mskb

    ln1 = _layer_norm(x, tng_ref[...], tnb_ref[...])
    y = ln1.reshape(_K, _B * _HID)
    tmid = _gelu(_dot(tl1t_ref[...], y) + tl1b_ref[...])
    tout = _dot(tl2t_ref[...], tmid) + tl2b_ref[...]
    h_token = tout.reshape(r, _HID) + x

    ln2 = _layer_norm(h_token, cng_ref[...], cnb_ref[...])
    u = _gelu(_dot(ln2, cl1_ref[...]) + cl1b_ref[...])
    v = _dot(u, cl2_ref[...]) + cl2b_ref[...]
    h_chan = v + h_token

    ln3 = _layer_norm(h_chan, hng_ref[...], hnb_ref[...])
    acc = ln3[0:_B, :]
    for k in range(1, _K):
        acc = acc + ln3[k * _B:(k + 1) * _B, :]
    mean = acc * np.float32(1.0 / _K)
    out_ref[...] = _dot(mean, hlw_ref[...]) + hlb_ref[...]


def _run_mixer(dt_t, msk_t, attr3, te_w, th_W, th_b, tn_g, tn_b, tl1_W,
               tl1_b, tl2_W, tl2_b, cn_g, cn_b, cl1_W, cl1_b, cl2_W, cl2_b,
               hn_g, hn_b, hl_W, hl_b):
    grid = (_NP // _B,)
    full = lambda shape: pl.BlockSpec(shape, lambda i: (0,) * len(shape))
    in_specs = [
        pl.BlockSpec((_K, _B), lambda i: (0, i)),
        pl.BlockSpec((_K, _B), lambda i: (0, i)),
        pl.BlockSpec((_K, _B, _IN), lambda i: (0, i, 0)),
        full((1, _TCH)),
        full((_TCH, _HID)),
        full((_IN, _HID)),
        full((1, _HID)),
        full((1, _HID)), full((1, _HID)),
        full((_K // 2, _K)), full((_K // 2, 1)),
        full((_K, _K // 2)), full((_K, 1)),
        full((1, _HID)), full((1, _HID)),
        full((_HID, 4 * _HID)), full((1, 4 * _HID)),
        full((4 * _HID, _HID)), full((1, _HID)),
        full((1, _HID)), full((1, _HID)),
        full((_HID, _OUT)), full((1, _OUT)),
    ]
    out = pl.pallas_call(
        _mixer_body,
        grid=grid,
        in_specs=in_specs,
        out_specs=pl.BlockSpec((_B, _OUT), lambda i: (i, 0)),
        out_shape=jax.ShapeDtypeStruct((_NP, _OUT), jnp.float32),
    )(dt_t, msk_t, attr3, te_w.reshape(1, _TCH),
      th_W[:_TCH], th_W[_TCH:], th_b.reshape(1, _HID),
      tn_g.reshape(1, _HID), tn_b.reshape(1, _HID),
      tl1_W.T, tl1_b.reshape(_K // 2, 1),
      tl2_W.T, tl2_b.reshape(_K, 1),
      cn_g.reshape(1, _HID), cn_b.reshape(1, _HID),
      cl1_W, cl1_b.reshape(1, 4 * _HID),
      cl2_W, cl2_b.reshape(1, _HID),
      hn_g.reshape(1, _HID), hn_b.reshape(1, _HID),
      hl_W, hl_b.reshape(1, _OUT))
    return out


def kernel(edge_index, edge_attr, edge_time, seed_time, th_W, th_b, tn_g,
           tn_b, tl1_W, tl1_b, tl2_W, tl2_b, cn_g, cn_b, cl1_W, cl1_b,
           cl2_W, cl2_b, hn_g, hn_b, hl_W, hl_b):
    col = edge_index[1]
    t = edge_time
    mask = t <= seed_time[col]
    col_eff = jnp.where(mask, col, _N).astype(jnp.int32)
    perm = jnp.lexsort([-t, col_eff]).astype(jnp.int32)
    cs = col_eff[perm]
    mask_s = mask[perm]
    col_s = col[perm]
    dt = (seed_time[col_s] - t[perm]).astype(jnp.float32)
    rank = (jnp.arange(_E, dtype=jnp.int32)
            - jnp.searchsorted(cs, cs, side="left").astype(jnp.int32))
    valid = mask_s & (rank < _K)
    slot = jnp.where(valid, rank * _NP + cs, _NP * _K)

    eid = jnp.zeros((_NP * _K + 1,), jnp.int32).at[slot].set(
        perm, mode="drop")[:_NP * _K]
    dtd = jnp.zeros((_NP * _K + 1,), jnp.float32).at[slot].set(
        dt, mode="drop")[:_NP * _K]
    mskd = jnp.zeros((_NP * _K + 1,), jnp.float32).at[slot].set(
        1.0, mode="drop")[:_NP * _K]
    attr_t = edge_attr[eid]

    te_w = (1.0 / 10.0 ** jnp.linspace(
        0.0, float(np.sqrt(_TCH)), _TCH)).astype(jnp.float32)

    out = _run_mixer(dtd.reshape(_K, _NP), mskd.reshape(_K, _NP),
                     attr_t.reshape(_K, _NP, _IN), te_w, th_W, th_b,
                     tn_g, tn_b, tl1_W, tl1_b, tl2_W, tl2_b, cn_g, cn_b,
                     cl1_W, cl1_b, cl2_W, cl2_b, hn_g, hn_b, hl_W, hl_b)
    return out[:_N]


# speedup vs baseline: 6.0670x; 5.3717x over previous
import functools

import numpy as np
import jax
import jax.numpy as jnp
from jax.experimental import pallas as pl
from jax.experimental.pallas import tpu as pltpu

_N = 10000
_E = 320000
_K = 32
_IN = 128
_HID = 256
_TCH = 128
_OUT = 256
_NP = 10240
_B = 128


def _layer_norm(x, g, b):
    m = jnp.mean(x, axis=-1, keepdims=True)
    v = jnp.mean((x - m) ** 2, axis=-1, keepdims=True)
    return (x - m) * jax.lax.rsqrt(v + 1e-5) * g + b


def _gelu(x):
    return x * 0.5 * (1.0 + jax.lax.erf(x * np.float32(0.7071067811865476)))


def _dot(a, b):
    return jnp.dot(a, b, preferred_element_type=jnp.float32)


def _mixer_body(dt_ref, msk_ref, attr_ref, tew_ref, thwt_ref, thwa_ref,
                thb_ref, tng_ref, tnb_ref, tl1t_ref, tl1b_ref, tl2t_ref,
                tl2b_ref, cng_ref, cnb_ref, cl1_ref, cl1b_ref, cl2_ref,
                cl2b_ref, hng_ref, hnb_ref, hlw_ref, hlb_ref, out_ref):
    r = _K * _B
    dt3 = dt_ref[...][:, :, None]
    msk3 = msk_ref[...][:, :, None]
    tew = tew_ref[...].reshape(1, 1, _TCH)
    te2 = jnp.cos(dt3 * tew).reshape(r, _TCH)
    attr2 = attr_ref[...].reshape(r, _IN)
    mskb = jnp.broadcast_to(msk3, (_K, _B, _HID)).reshape(r, _HID)

    h = _dot(te2, thwt_ref[...]) + _dot(attr2, thwa_ref[...]) + thb_ref[...]
    x = h * mskb

    ln1 = _layer_norm(x, tng_ref[...], tnb_ref[...])
    y = ln1.reshape(_K, _B * _HID)
    tmid = _gelu(_dot(tl1t_ref[...], y) + tl1b_ref[...])
    tout = _dot(tl2t_ref[...], tmid) + tl2b_ref[...]
    h_token = tout.reshape(r, _HID) + x

    ln2 = _layer_norm(h_token, cng_ref[...], cnb_ref[...])
    u = _gelu(_dot(ln2, cl1_ref[...]) + cl1b_ref[...])
    v = _dot(u, cl2_ref[...]) + cl2b_ref[...]
    h_chan = v + h_token

    ln3 = _layer_norm(h_chan, hng_ref[...], hnb_ref[...])
    acc = ln3[0:_B, :]
    for k in range(1, _K):
        acc = acc + ln3[k * _B:(k + 1) * _B, :]
    mean = acc * np.float32(1.0 / _K)
    out_ref[...] = _dot(mean, hlw_ref[...]) + hlb_ref[...]


def _run_mixer(dt_t, msk_t, attr3, te_w, th_W, th_b, tn_g, tn_b, tl1_W,
               tl1_b, tl2_W, tl2_b, cn_g, cn_b, cl1_W, cl1_b, cl2_W, cl2_b,
               hn_g, hn_b, hl_W, hl_b):
    grid = (_NP // _B,)
    full = lambda shape: pl.BlockSpec(shape, lambda i: (0,) * len(shape))
    in_specs = [
        pl.BlockSpec((_K, _B), lambda i: (0, i)),
        pl.BlockSpec((_K, _B), lambda i: (0, i)),
        pl.BlockSpec((_K, _B, _IN), lambda i: (0, i, 0)),
        full((1, _TCH)),
        full((_TCH, _HID)),
        full((_IN, _HID)),
        full((1, _HID)),
        full((1, _HID)), full((1, _HID)),
        full((_K // 2, _K)), full((_K // 2, 1)),
        full((_K, _K // 2)), full((_K, 1)),
        full((1, _HID)), full((1, _HID)),
        full((_HID, 4 * _HID)), full((1, 4 * _HID)),
        full((4 * _HID, _HID)), full((1, _HID)),
        full((1, _HID)), full((1, _HID)),
        full((_HID, _OUT)), full((1, _OUT)),
    ]
    out = pl.pallas_call(
        _mixer_body,
        grid=grid,
        in_specs=in_specs,
        out_specs=pl.BlockSpec((_B, _OUT), lambda i: (i, 0)),
        out_shape=jax.ShapeDtypeStruct((_NP, _OUT), jnp.float32),
    )(dt_t, msk_t, attr3, te_w.reshape(1, _TCH),
      th_W[:_TCH], th_W[_TCH:], th_b.reshape(1, _HID),
      tn_g.reshape(1, _HID), tn_b.reshape(1, _HID),
      tl1_W.T, tl1_b.reshape(_K // 2, 1),
      tl2_W.T, tl2_b.reshape(_K, 1),
      cn_g.reshape(1, _HID), cn_b.reshape(1, _HID),
      cl1_W, cl1_b.reshape(1, 4 * _HID),
      cl2_W, cl2_b.reshape(1, _HID),
      hn_g.reshape(1, _HID), hn_b.reshape(1, _HID),
      hl_W, hl_b.reshape(1, _OUT))
    return out


def kernel(edge_index, edge_attr, edge_time, seed_time, th_W, th_b, tn_g,
           tn_b, tl1_W, tl1_b, tl2_W, tl2_b, cn_g, cn_b, cl1_W, cl1_b,
           cl2_W, cl2_b, hn_g, hn_b, hl_W, hl_b):
    col = edge_index[1]
    t = edge_time
    st_col = seed_time[col]
    mask = t <= st_col
    col_eff = jnp.where(mask, col, _N).astype(jnp.int32)
    iota = jnp.arange(_E, dtype=jnp.int32)
    cs, _, dt_s, eid_s = jax.lax.sort(
        (col_eff, -t, (st_col - t).astype(jnp.float32), iota), num_keys=2)
    start = jnp.searchsorted(
        cs, jnp.arange(_N + 1, dtype=jnp.int32)).astype(jnp.int32)
    cnt = start[1:] - start[:-1]
    start_n = jnp.pad(start[:-1], (0, _NP - _N), constant_values=_E)
    cnt_n = jnp.pad(cnt, (0, _NP - _N))
    karr = jnp.arange(_K, dtype=jnp.int32)
    pos = start_n[None, :] + karr[:, None]
    valid = karr[:, None] < jnp.minimum(cnt_n, _K)[None, :]
    posc = jnp.minimum(pos, _E - 1).reshape(-1)
    eid = eid_s[posc]
    dtd = jnp.where(valid.reshape(-1), dt_s[posc], 0.0)
    mskd = valid.reshape(-1).astype(jnp.float32)
    attr_t = edge_attr[eid]

    te_w = (1.0 / 10.0 ** jnp.linspace(
        0.0, float(np.sqrt(_TCH)), _TCH)).astype(jnp.float32)

    out = _run_mixer(dtd.reshape(_K, _NP), mskd.reshape(_K, _NP),
                     attr_t.reshape(_K, _NP, _IN), te_w, th_W, th_b,
                     tn_g, tn_b, tl1_W, tl1_b, tl2_W, tl2_b, cn_g, cn_b,
                     cl1_W, cl1_b, cl2_W, cl2_b, hn_g, hn_b, hl_W, hl_b)
    return out[:_N]


# speedup vs baseline: 6.0815x; 1.0024x over previous
import functools

import numpy as np
import jax
import jax.numpy as jnp
from jax.experimental import pallas as pl
from jax.experimental.pallas import tpu as pltpu

_N = 10000
_E = 320000
_K = 32
_IN = 128
_HID = 256
_TCH = 128
_OUT = 256
_NP = 10240
_B = 128


def _layer_norm(x, g, b):
    m = jnp.mean(x, axis=-1, keepdims=True)
    v = jnp.mean((x - m) ** 2, axis=-1, keepdims=True)
    return (x - m) * jax.lax.rsqrt(v + 1e-5) * g + b


def _gelu(x):
    return x * 0.5 * (1.0 + jax.lax.erf(x * np.float32(0.7071067811865476)))


def _dot(a, b):
    return jnp.dot(a.astype(jnp.bfloat16), b.astype(jnp.bfloat16),
                   preferred_element_type=jnp.float32)


def _mixer_body(dt_ref, msk_ref, attr_ref, tew_ref, thwt_ref, thwa_ref,
                thb_ref, tng_ref, tnb_ref, tl1t_ref, tl1b_ref, tl2t_ref,
                tl2b_ref, cng_ref, cnb_ref, cl1_ref, cl1b_ref, cl2_ref,
                cl2b_ref, hng_ref, hnb_ref, hlw_ref, hlb_ref, out_ref):
    r = _K * _B
    dt3 = dt_ref[...][:, :, None]
    msk3 = msk_ref[...][:, :, None]
    tew = tew_ref[...].reshape(1, 1, _TCH)
    te2 = jnp.cos(dt3 * tew).reshape(r, _TCH)
    attr2 = attr_ref[...].reshape(r, _IN)
    mskb = jnp.broadcast_to(msk3, (_K, _B, _HID)).reshape(r, _HID)

    h = _dot(te2, thwt_ref[...]) + _dot(attr2, thwa_ref[...]) + thb_ref[...]
    x = h * mskb

    ln1 = _layer_norm(x, tng_ref[...], tnb_ref[...])
    y = ln1.reshape(_K, _B * _HID)
    tmid = _gelu(_dot(tl1t_ref[...], y) + tl1b_ref[...])
    tout = _dot(tl2t_ref[...], tmid) + tl2b_ref[...]
    h_token = tout.reshape(r, _HID) + x

    ln2 = _layer_norm(h_token, cng_ref[...], cnb_ref[...])
    u = _gelu(_dot(ln2, cl1_ref[...]) + cl1b_ref[...])
    v = _dot(u, cl2_ref[...]) + cl2b_ref[...]
    h_chan = v + h_token

    ln3 = _layer_norm(h_chan, hng_ref[...], hnb_ref[...])
    acc = ln3[0:_B, :]
    for k in range(1, _K):
        acc = acc + ln3[k * _B:(k + 1) * _B, :]
    mean = acc * np.float32(1.0 / _K)
    out_ref[...] = _dot(mean, hlw_ref[...]) + hlb_ref[...]


def _run_mixer(dt_t, msk_t, attr3, te_w, th_W, th_b, tn_g, tn_b, tl1_W,
               tl1_b, tl2_W, tl2_b, cn_g, cn_b, cl1_W, cl1_b, cl2_W, cl2_b,
               hn_g, hn_b, hl_W, hl_b):
    grid = (_NP // _B,)
    full = lambda shape: pl.BlockSpec(shape, lambda i: (0,) * len(shape))
    in_specs = [
        pl.BlockSpec((_K, _B), lambda i: (0, i)),
        pl.BlockSpec((_K, _B), lambda i: (0, i)),
        pl.BlockSpec((_K, _B, _IN), lambda i: (0, i, 0)),
        full((1, _TCH)),
        full((_TCH, _HID)),
        full((_IN, _HID)),
        full((1, _HID)),
        full((1, _HID)), full((1, _HID)),
        full((_K // 2, _K)), full((_K // 2, 1)),
        full((_K, _K // 2)), full((_K, 1)),
        full((1, _HID)), full((1, _HID)),
        full((_HID, 4 * _HID)), full((1, 4 * _HID)),
        full((4 * _HID, _HID)), full((1, _HID)),
        full((1, _HID)), full((1, _HID)),
        full((_HID, _OUT)), full((1, _OUT)),
    ]
    out = pl.pallas_call(
        _mixer_body,
        grid=grid,
        in_specs=in_specs,
        out_specs=pl.BlockSpec((_B, _OUT), lambda i: (i, 0)),
        out_shape=jax.ShapeDtypeStruct((_NP, _OUT), jnp.float32),
    )(dt_t, msk_t, attr3, te_w.reshape(1, _TCH),
      th_W[:_TCH], th_W[_TCH:], th_b.reshape(1, _HID),
      tn_g.reshape(1, _HID), tn_b.reshape(1, _HID),
      tl1_W.T, tl1_b.reshape(_K // 2, 1),
      tl2_W.T, tl2_b.reshape(_K, 1),
      cn_g.reshape(1, _HID), cn_b.reshape(1, _HID),
      cl1_W, cl1_b.reshape(1, 4 * _HID),
      cl2_W, cl2_b.reshape(1, _HID),
      hn_g.reshape(1, _HID), hn_b.reshape(1, _HID),
      hl_W, hl_b.reshape(1, _OUT))
    return out


def kernel(edge_index, edge_attr, edge_time, seed_time, th_W, th_b, tn_g,
           tn_b, tl1_W, tl1_b, tl2_W, tl2_b, cn_g, cn_b, cl1_W, cl1_b,
           cl2_W, cl2_b, hn_g, hn_b, hl_W, hl_b):
    col = edge_index[1]
    t = edge_time
    st_col = seed_time[col]
    mask = t <= st_col
    col_eff = jnp.where(mask, col, _N).astype(jnp.int32)
    iota = jnp.arange(_E, dtype=jnp.int32)
    cs, _, dt_s, eid_s = jax.lax.sort(
        (col_eff, -t, (st_col - t).astype(jnp.float32), iota), num_keys=2)
    start = jnp.searchsorted(
        cs, jnp.arange(_N + 1, dtype=jnp.int32)).astype(jnp.int32)
    cnt = start[1:] - start[:-1]
    start_n = jnp.pad(start[:-1], (0, _NP - _N), constant_values=_E)
    cnt_n = jnp.pad(cnt, (0, _NP - _N))
    karr = jnp.arange(_K, dtype=jnp.int32)
    pos = start_n[None, :] + karr[:, None]
    valid = karr[:, None] < jnp.minimum(cnt_n, _K)[None, :]
    posc = jnp.minimum(pos, _E - 1).reshape(-1)
    eid = eid_s[posc]
    dtd = jnp.where(valid.reshape(-1), dt_s[posc], 0.0)
    mskd = valid.reshape(-1).astype(jnp.float32)
    attr_t = edge_attr[eid]

    te_w = (1.0 / 10.0 ** jnp.linspace(
        0.0, float(np.sqrt(_TCH)), _TCH)).astype(jnp.float32)

    out = _run_mixer(dtd.reshape(_K, _NP), mskd.reshape(_K, _NP),
                     attr_t.reshape(_K, _NP, _IN), te_w, th_W, th_b,
                     tn_g, tn_b, tl1_W, tl1_b, tl2_W, tl2_b, cn_g, cn_b,
                     cl1_W, cl1_b, cl2_W, cl2_b, hn_g, hn_b, hl_W, hl_b)
    return out[:_N]
